# Initial kernel scaffold; baseline (speedup 1.0000x reference)
import jax
import jax.numpy as jnp
from jax.experimental import pallas as pl


def kernel(xy, grid, scale, offset):
    raise NotImplementedError("write your pallas kernel here")


# speedup vs baseline: 1.4610x; 1.4610x over previous
import functools

import jax
import jax.numpy as jnp
from jax import lax
from jax.experimental import pallas as pl
from jax.experimental.pallas import tpu as pltpu
from jax.experimental.pallas import tpu_sc as plsc

GRID_H = 1024
GRID_W = 1024
BATCH = 16384
LANES = 16

_info = plsc.get_sparse_core_info()
_NC = _info.num_cores
_NS = _info.num_subcores
_NW = _NC * _NS
_PTS = BATCH // _NW
_CHUNK = 128
_NCHUNK = _PTS // _CHUNK
_VPC = _CHUNK // LANES


def _sc_body(x_hbm, y_hbm, grid_hbm, sv_hbm, ov_hbm, out_hbm,
             xv, yv, wxv, wyv, outv, sv, ov,
             i00, i01, i10, i11, g00, g01, g10, g11, sem):
    wid = lax.axis_index("s") * _NC + lax.axis_index("c")
    base = wid * _PTS
    pltpu.sync_copy(x_hbm.at[pl.ds(base, _PTS)], xv)
    pltpu.sync_copy(y_hbm.at[pl.ds(base, _PTS)], yv)
    pltpu.sync_copy(sv_hbm, sv)
    pltpu.sync_copy(ov_hbm, ov)

    copies = []
    for c in range(_NCHUNK):
        for g in range(_VPC):
            i = c * _CHUNK + g * LANES
            xx = xv[pl.ds(i, LANES)]
            yy = yv[pl.ds(i, LANES)]
            xf = jnp.minimum(jnp.maximum(xx, 0.0), 1.0) * float(GRID_W - 1)
            yf = jnp.minimum(jnp.maximum(yy, 0.0), 1.0) * float(GRID_H - 1)
            x0 = xf.astype(jnp.int32)
            y0 = yf.astype(jnp.int32)
            wx = xf - x0.astype(jnp.float32)
            wy = yf - y0.astype(jnp.float32)
            dx = jnp.minimum(x0 + 1, GRID_W - 1) - x0
            r0 = y0 * GRID_W + x0
            r1 = jnp.minimum(y0 + 1, GRID_H - 1) * GRID_W + x0
            s = pl.ds(g * LANES, LANES)
            i00[c, s] = r0
            i01[c, s] = r0 + dx
            i10[c, s] = r1
            i11[c, s] = r1 + dx
            wxv[pl.ds(i, LANES)] = wx
            wyv[pl.ds(i, LANES)] = wy
        copies.append(pltpu.async_copy(grid_hbm.at[i00.at[c]], g00.at[c], sem))
        copies.append(pltpu.async_copy(grid_hbm.at[i01.at[c]], g01.at[c], sem))
        copies.append(pltpu.async_copy(grid_hbm.at[i10.at[c]], g10.at[c], sem))
        copies.append(pltpu.async_copy(grid_hbm.at[i11.at[c]], g11.at[c], sem))
    for cp in copies:
        cp.wait()

    sc = sv[...]
    of = ov[...]
    for c in range(_NCHUNK):
        for g in range(_VPC):
            i = c * _CHUNK + g * LANES
            s = pl.ds(g * LANES, LANES)
            a00 = g00[c, s]
            a01 = g01[c, s]
            a10 = g10[c, s]
            a11 = g11[c, s]
            wx = wxv[pl.ds(i, LANES)]
            wy = wyv[pl.ds(i, LANES)]
            top = a00 + wx * (a01 - a00)
            bot = a10 + wx * (a11 - a10)
            val = top + wy * (bot - top)
            outv[pl.ds(i, LANES)] = val * sc + of
    pltpu.sync_copy(outv, out_hbm.at[pl.ds(base, _PTS)])


_bilinear_sc = functools.partial(
    pl.kernel,
    out_type=jax.ShapeDtypeStruct((BATCH,), jnp.float32),
    mesh=plsc.VectorSubcoreMesh(core_axis_name="c", subcore_axis_name="s"),
    scratch_types=[
        pltpu.VMEM((_PTS,), jnp.float32),
        pltpu.VMEM((_PTS,), jnp.float32),
        pltpu.VMEM((_PTS,), jnp.float32),
        pltpu.VMEM((_PTS,), jnp.float32),
        pltpu.VMEM((_PTS,), jnp.float32),
        pltpu.VMEM((LANES,), jnp.float32),
        pltpu.VMEM((LANES,), jnp.float32),
        pltpu.VMEM((_NCHUNK, _CHUNK), jnp.int32),
        pltpu.VMEM((_NCHUNK, _CHUNK), jnp.int32),
        pltpu.VMEM((_NCHUNK, _CHUNK), jnp.int32),
        pltpu.VMEM((_NCHUNK, _CHUNK), jnp.int32),
        pltpu.VMEM((_NCHUNK, _CHUNK), jnp.float32),
        pltpu.VMEM((_NCHUNK, _CHUNK), jnp.float32),
        pltpu.VMEM((_NCHUNK, _CHUNK), jnp.float32),
        pltpu.VMEM((_NCHUNK, _CHUNK), jnp.float32),
        pltpu.SemaphoreType.DMA,
    ],
)(_sc_body)


def kernel(xy, grid, scale, offset):
    x = xy[:, 0]
    y = xy[:, 1]
    gflat = grid.reshape(-1)
    sv = jnp.broadcast_to(jnp.asarray(scale, jnp.float32), (LANES,))
    ov = jnp.broadcast_to(jnp.asarray(offset, jnp.float32), (LANES,))
    return _bilinear_sc(x, y, gflat, sv, ov)
